# baseline (speedup 1.0000x reference)
import functools

import jax
import jax.numpy as jnp
from jax import lax
from jax.experimental import pallas as pl
from jax.experimental.pallas import tpu as pltpu
from jax.experimental.pallas import tpu_sc as plsc

_T = 4096
_D = 256
_K = 8192
_TB = 512
_KC = 1024
_NTB = _T // _TB
_NKC = _K // _KC

_NC = 2
_NS = 16
_NW = _NC * _NS
_BW = _T // _NW


def _argmin_body(r_ref, cbT_ref, csq_ref, out_ref, minv, mini):
    kc = pl.program_id(1)
    cross = jnp.dot(r_ref[...], cbT_ref[...],
                    preferred_element_type=jnp.float32,
                    precision=lax.Precision.DEFAULT)
    t = csq_ref[...] - 2.0 * cross
    mloc = jnp.min(t, axis=1, keepdims=True)
    iota = lax.broadcasted_iota(jnp.int32, t.shape, 1)
    iloc = jnp.min(jnp.where(t == mloc, iota, _K), axis=1, keepdims=True)
    iloc = iloc + kc * _KC

    @pl.when(kc == 0)
    def _():
        minv[...] = mloc
        mini[...] = iloc

    @pl.when(kc > 0)
    def _():
        better = mloc < minv[...]
        minv[...] = jnp.where(better, mloc, minv[...])
        mini[...] = jnp.where(better, iloc, mini[...])

    @pl.when(kc == _NKC - 1)
    def _():
        out_ref[...] = mini[...]


def _tc_argmin(r, cbT, csq):
    return pl.pallas_call(
        _argmin_body,
        grid=(_NTB, _NKC),
        in_specs=[
            pl.BlockSpec((_TB, _D), lambda tb, kc: (tb, 0)),
            pl.BlockSpec((_D, _KC), lambda tb, kc: (0, kc)),
            pl.BlockSpec((1, _KC), lambda tb, kc: (0, kc)),
        ],
        out_specs=pl.BlockSpec((_TB, 1), lambda tb, kc: (tb, 0)),
        out_shape=jax.ShapeDtypeStruct((_T, 1), jnp.int32),
        scratch_shapes=[pltpu.VMEM((_TB, 1), jnp.float32),
                        pltpu.VMEM((_TB, 1), jnp.int32)],
    )(r, cbT, csq)


def _sc_mesh():
    return plsc.VectorSubcoreMesh(core_axis_name="c", subcore_axis_name="s",
                                  num_cores=_NC, num_subcores=_NS)


def _sc_residual_update(r, idx, cb):
    @functools.partial(
        pl.kernel,
        out_type=jax.ShapeDtypeStruct((_T, _D), jnp.float32),
        mesh=_sc_mesh(),
        scratch_types=[pltpu.VMEM((_BW,), jnp.int32),
                       pltpu.VMEM((_BW, _D), jnp.float32),
                       pltpu.VMEM((_BW, _D), jnp.float32),
                       pltpu.SemaphoreType.DMA],
    )
    def k(r_hbm, idx_hbm, cb_hbm, out_hbm, idx_v, rows_v, r_v, sem):
        wid = lax.axis_index("s") * _NC + lax.axis_index("c")
        base = wid * _BW
        pltpu.sync_copy(idx_hbm.at[pl.ds(base, _BW)], idx_v)
        cp = pltpu.async_copy(cb_hbm.at[idx_v], rows_v, sem)
        pltpu.sync_copy(r_hbm.at[pl.ds(base, _BW)], r_v)
        cp.wait()

        def row_fn(i, carry):
            for j in range(_D // 16):
                s = pl.ds(j * 16, 16)
                rows_v[i, s] = r_v[i, s] - rows_v[i, s]
            return carry

        lax.fori_loop(0, _BW, row_fn, 0)
        pltpu.sync_copy(rows_v, out_hbm.at[pl.ds(base, _BW)])

    return k(r, idx, cb)


def _sc_final_output(r, idx, cb, h):
    @functools.partial(
        pl.kernel,
        out_type=jax.ShapeDtypeStruct((_T, _D), jnp.float32),
        mesh=_sc_mesh(),
        scratch_types=[pltpu.VMEM((_BW,), jnp.int32),
                       pltpu.VMEM((_BW, _D), jnp.float32),
                       pltpu.VMEM((_BW, _D), jnp.float32),
                       pltpu.VMEM((_BW, _D), jnp.float32),
                       pltpu.SemaphoreType.DMA],
    )
    def k(r_hbm, idx_hbm, cb_hbm, h_hbm, out_hbm, idx_v, rows_v, r_v, h_v, sem):
        wid = lax.axis_index("s") * _NC + lax.axis_index("c")
        base = wid * _BW
        pltpu.sync_copy(idx_hbm.at[pl.ds(base, _BW)], idx_v)
        cp = pltpu.async_copy(cb_hbm.at[idx_v], rows_v, sem)
        pltpu.sync_copy(r_hbm.at[pl.ds(base, _BW)], r_v)
        pltpu.sync_copy(h_hbm.at[pl.ds(base, _BW)], h_v)
        cp.wait()

        def row_fn(i, carry):
            for j in range(_D // 16):
                s = pl.ds(j * 16, 16)
                rows_v[i, s] = (h_v[i, s] - r_v[i, s]) + rows_v[i, s]
            return carry

        lax.fori_loop(0, _BW, row_fn, 0)
        pltpu.sync_copy(rows_v, out_hbm.at[pl.ds(base, _BW)])

    return k(r, idx, cb, h)


def kernel(x, codebooks):
    b, d, n = x.shape
    num_q = codebooks.shape[0]
    h = jnp.transpose(x, (0, 2, 1)).reshape(b * n, d)
    cbT = jnp.transpose(codebooks, (0, 2, 1))
    csq = jnp.sum(codebooks * codebooks, axis=-1)

    r = h
    out_tok = None
    for q in range(num_q):
        idx = _tc_argmin(r, cbT[q], csq[q][None, :])[:, 0]
        if q < num_q - 1:
            r = _sc_residual_update(r, idx, codebooks[q])
        else:
            out_tok = _sc_final_output(r, idx, codebooks[q], h)
    return jnp.transpose(out_tok.reshape(b, n, d), (0, 2, 1))


# speedup vs baseline: 1.0590x; 1.0590x over previous
import functools

import jax
import jax.numpy as jnp
from jax import lax
from jax.experimental import pallas as pl
from jax.experimental.pallas import tpu as pltpu
from jax.experimental.pallas import tpu_sc as plsc

_T = 4096
_D = 256
_K = 8192
_TB = 512
_KC = 1024
_NTB = _T // _TB
_NKC = _K // _KC

_NC = 2
_NS = 16
_NW = _NC * _NS
_BW = _T // _NW


def _argmin_body(r_ref, cb_ref, csq_ref, out_ref, minv, mini):
    kc = pl.program_id(1)
    cross = lax.dot_general(r_ref[...], cb_ref[...],
                            (((1,), (1,)), ((), ())),
                            preferred_element_type=jnp.float32,
                            precision=lax.Precision.DEFAULT)
    t = csq_ref[...] - 2.0 * cross
    mloc = jnp.min(t, axis=1, keepdims=True)
    iota = lax.broadcasted_iota(jnp.int32, t.shape, 1)
    iloc = jnp.min(jnp.where(t == mloc, iota, _K), axis=1, keepdims=True)
    iloc = iloc + kc * _KC

    @pl.when(kc == 0)
    def _():
        minv[...] = mloc
        mini[...] = iloc

    @pl.when(kc > 0)
    def _():
        better = mloc < minv[...]
        minv[...] = jnp.where(better, mloc, minv[...])
        mini[...] = jnp.where(better, iloc, mini[...])

    @pl.when(kc == _NKC - 1)
    def _():
        out_ref[...] = mini[...]


def _tc_argmin(r, cb, csq):
    return pl.pallas_call(
        _argmin_body,
        grid=(_NTB, _NKC),
        in_specs=[
            pl.BlockSpec((_TB, _D), lambda tb, kc: (tb, 0)),
            pl.BlockSpec((_KC, _D), lambda tb, kc: (kc, 0)),
            pl.BlockSpec((1, _KC), lambda tb, kc: (0, kc)),
        ],
        out_specs=pl.BlockSpec((_TB, 1), lambda tb, kc: (tb, 0)),
        out_shape=jax.ShapeDtypeStruct((_T, 1), jnp.int32),
        scratch_shapes=[pltpu.VMEM((_TB, 1), jnp.float32),
                        pltpu.VMEM((_TB, 1), jnp.int32)],
    )(r, cb, csq)


def _sc_mesh():
    return plsc.VectorSubcoreMesh(core_axis_name="c", subcore_axis_name="s",
                                  num_cores=_NC, num_subcores=_NS)


def _sc_residual_update(r, idx, cb):
    @functools.partial(
        pl.kernel,
        out_type=jax.ShapeDtypeStruct((_T, _D), jnp.float32),
        mesh=_sc_mesh(),
        scratch_types=[pltpu.VMEM((_BW,), jnp.int32),
                       pltpu.VMEM((_BW, _D), jnp.float32),
                       pltpu.VMEM((_BW, _D), jnp.float32),
                       pltpu.SemaphoreType.DMA],
    )
    def k(r_hbm, idx_hbm, cb_hbm, out_hbm, idx_v, rows_v, r_v, sem):
        wid = lax.axis_index("s") * _NC + lax.axis_index("c")
        base = wid * _BW
        pltpu.sync_copy(idx_hbm.at[pl.ds(base, _BW)], idx_v)
        cp = pltpu.async_copy(cb_hbm.at[idx_v], rows_v, sem)
        pltpu.sync_copy(r_hbm.at[pl.ds(base, _BW)], r_v)
        cp.wait()

        def row_fn(i, carry):
            for j in range(_D // 16):
                s = pl.ds(j * 16, 16)
                rows_v[i, s] = r_v[i, s] - rows_v[i, s]
            return carry

        lax.fori_loop(0, _BW, row_fn, 0)
        pltpu.sync_copy(rows_v, out_hbm.at[pl.ds(base, _BW)])

    return k(r, idx, cb)


def _sc_final_output(r, idx, cb, h):
    @functools.partial(
        pl.kernel,
        out_type=jax.ShapeDtypeStruct((_T, _D), jnp.float32),
        mesh=_sc_mesh(),
        scratch_types=[pltpu.VMEM((_BW,), jnp.int32),
                       pltpu.VMEM((_BW, _D), jnp.float32),
                       pltpu.VMEM((_BW, _D), jnp.float32),
                       pltpu.VMEM((_BW, _D), jnp.float32),
                       pltpu.SemaphoreType.DMA],
    )
    def k(r_hbm, idx_hbm, cb_hbm, h_hbm, out_hbm, idx_v, rows_v, r_v, h_v, sem):
        wid = lax.axis_index("s") * _NC + lax.axis_index("c")
        base = wid * _BW
        pltpu.sync_copy(idx_hbm.at[pl.ds(base, _BW)], idx_v)
        cp = pltpu.async_copy(cb_hbm.at[idx_v], rows_v, sem)
        pltpu.sync_copy(r_hbm.at[pl.ds(base, _BW)], r_v)
        pltpu.sync_copy(h_hbm.at[pl.ds(base, _BW)], h_v)
        cp.wait()

        def row_fn(i, carry):
            for j in range(_D // 16):
                s = pl.ds(j * 16, 16)
                rows_v[i, s] = (h_v[i, s] - r_v[i, s]) + rows_v[i, s]
            return carry

        lax.fori_loop(0, _BW, row_fn, 0)
        pltpu.sync_copy(rows_v, out_hbm.at[pl.ds(base, _BW)])

    return k(r, idx, cb, h)


def kernel(x, codebooks):
    b, d, n = x.shape
    num_q = codebooks.shape[0]
    h = jnp.transpose(x, (0, 2, 1)).reshape(b * n, d)
    csq = jnp.sum(codebooks * codebooks, axis=-1)

    r = h
    out_tok = None
    for q in range(num_q):
        idx = _tc_argmin(r, codebooks[q], csq[q][None, :])[:, 0]
        if q < num_q - 1:
            r = _sc_residual_update(r, idx, codebooks[q])
        else:
            out_tok = _sc_final_output(r, idx, codebooks[q], h)
    return jnp.transpose(out_tok.reshape(b, n, d), (0, 2, 1))
